# baseline (speedup 1.0000x reference)
import functools

import jax
import jax.numpy as jnp
from jax.experimental import pallas as pl
from jax.experimental.pallas import tpu as pltpu


_BLK = 4000
_NPART = 4
_WIN = 24
_OUT_S = 1024


def _pool_kernel(*refs, nparts, psteps, s_out, win):
    blkinfo_ref, b2_ref = refs[0], refs[1]
    batch_refs = refs[2:2 + nparts]
    x_refs = refs[2 + nparts:2 + 2 * nparts]
    w1_ref, b1_ref, w2_ref = refs[2 + 2 * nparts:5 + 2 * nparts]
    out_ref = refs[5 + 2 * nparts]
    scratch = refs[6 + 2 * nparts:]
    acc_refs = scratch[0:nparts]
    m_refs = scratch[nparts:2 * nparts]
    l_refs = scratch[2 * nparts:3 * nparts]

    k = pl.program_id(0)

    @pl.when(k == 0)
    def _init():
        for q in range(nparts):
            acc_refs[q][:, :] = jnp.zeros_like(acc_refs[q])
            m_refs[q][:, :] = jnp.zeros_like(m_refs[q])
            l_refs[q][:, :] = jnp.zeros_like(l_refs[q])

    w1_bf = w1_ref[:, :].astype(jnp.bfloat16)
    w2_bf = w2_ref[:, :].astype(jnp.bfloat16)
    b1_bf = b1_ref[:, :].astype(jnp.bfloat16)
    b2 = b2_ref[0]

    x_bfs, gates = [], []
    for q in range(nparts):
        x_bf = x_refs[q][:, :].astype(jnp.bfloat16)
        h = jax.lax.dot_general(x_bf, w1_bf, (((1,), (1,)), ((), ())),
                                preferred_element_type=jnp.float32)
        h = jnp.maximum(h.astype(jnp.bfloat16) + b1_bf, 0)
        gate = jax.lax.dot_general(w2_bf, h, (((1,), (1,)), ((), ())),
                                   preferred_element_type=jnp.float32)
        gates.append(gate + b2)
        x_bfs.append(x_bf)

    idss = [batch_refs[q][0] for q in range(nparts)]
    start0s = [blkinfo_ref[q * psteps + k, 0] for q in range(nparts)]
    nwins = [(blkinfo_ref[q * psteps + k, 1] - start0s[q]) // win + 1
             for q in range(nparts)]
    nwin_max = nwins[0]
    for q in range(1, nparts):
        nwin_max = jnp.maximum(nwin_max, nwins[q])

    def body(j, carry):
        for q in range(nparts):
            @pl.when(j < nwins[q])
            def _do(q=q):
                ids, gate, x_bf = idss[q], gates[q], x_bfs[q]
                acc_ref, m_ref, l_ref = acc_refs[q], m_refs[q], l_refs[q]
                start = start0s[q] + j * win
                rows = start + jax.lax.broadcasted_iota(
                    jnp.int32, (win, 1), 0)
                hit = ids == rows
                bmax = jnp.max(jnp.where(hit, gate, -1e30), axis=1,
                               keepdims=True)
                m_old = m_ref[pl.ds(start, win), :]
                m_new = jnp.maximum(m_old, bmax)
                alpha = jnp.exp(m_old - m_new)
                m_ref[pl.ds(start, win), :] = m_new
                m_row = jnp.sum(jnp.where(hit, m_new, 0.0), axis=0,
                                keepdims=True)
                in_w = (ids >= start) & (ids < start + win)
                e = jnp.where(in_w, jnp.exp(gate - m_row), 0.0)
                p = jnp.where(hit, e, 0.0)
                l_add = jnp.sum(p, axis=1, keepdims=True)
                l_ref[pl.ds(start, win), :] = (
                    l_ref[pl.ds(start, win), :] * alpha + l_add)
                contrib = jax.lax.dot_general(
                    p.astype(jnp.bfloat16), x_bf, (((1,), (0,)), ((), ())),
                    preferred_element_type=jnp.float32)
                acc_ref[pl.ds(start, win), :] = (
                    acc_ref[pl.ds(start, win), :] * alpha + contrib)
        return carry

    jax.lax.fori_loop(0, nwin_max, body, 0)

    @pl.when(k == psteps - 1)
    def _fin():
        m_parts = [m_refs[q][0:s_out, :] for q in range(nparts)]
        m_all = m_parts[0]
        for q in range(1, nparts):
            m_all = jnp.maximum(m_all, m_parts[q])
        num = jnp.zeros((s_out, out_ref.shape[1]), jnp.float32)
        den = jnp.zeros((s_out, 1), jnp.float32)
        for q in range(nparts):
            scale = jnp.exp(m_parts[q] - m_all)
            num = num + acc_refs[q][0:s_out, :] * scale
            den = den + l_refs[q][0:s_out, :] * scale
        out_ref[:, :] = num / (den + 1e-9)


def _run(x, batch, W1, b1, W2, b2, s_out, blk, nparts, win, interpret=False):
    n, d = x.shape
    hdim = W1.shape[0]
    nsteps = n // blk
    psteps = nsteps // nparts
    assert psteps * nparts * blk == n
    s_pad = s_out + 2 * win

    batch3 = batch.reshape(nsteps, 1, blk)
    first = (batch3[:, 0, 0] // 8) * 8
    last = batch3[:, 0, blk - 1]
    blkinfo = jnp.stack([first, last], axis=1).astype(jnp.int32)

    def batch_map(q):
        return lambda k: (q * psteps + k, 0, 0)

    def x_map(q):
        return lambda k: (q * psteps + k, 0)

    in_specs = (
        [pl.BlockSpec(memory_space=pltpu.SMEM),
         pl.BlockSpec(memory_space=pltpu.SMEM)]
        + [pl.BlockSpec((1, 1, blk), batch_map(q)) for q in range(nparts)]
        + [pl.BlockSpec((blk, d), x_map(q)) for q in range(nparts)]
        + [pl.BlockSpec((hdim, d), lambda k: (0, 0)),
           pl.BlockSpec((1, hdim), lambda k: (0, 0)),
           pl.BlockSpec((1, hdim), lambda k: (0, 0))]
    )

    scratch_shapes = (
        [pltpu.VMEM((s_pad, d), jnp.float32) for _ in range(nparts)]
        + [pltpu.VMEM((s_pad, 1), jnp.float32) for _ in range(2 * nparts)]
    )

    fn = pl.pallas_call(
        functools.partial(_pool_kernel, nparts=nparts, psteps=psteps,
                          s_out=s_out, win=win),
        grid=(psteps,),
        in_specs=in_specs,
        out_specs=pl.BlockSpec((s_out, d), lambda k: (0, 0)),
        out_shape=jax.ShapeDtypeStruct((s_out, d), x.dtype),
        scratch_shapes=scratch_shapes,
        compiler_params=pltpu.CompilerParams(
            dimension_semantics=("arbitrary",)),
        interpret=interpret,
    )
    args = ([blkinfo, b2] + [batch3] * nparts + [x] * nparts
            + [W1, b1.reshape(1, hdim), W2])
    return fn(*args)


def kernel(x, batch, W1, b1, W2, b2):
    return _run(x, batch, W1, b1, W2, b2, _OUT_S, _BLK, _NPART, _WIN)


# speedup vs baseline: 1.1649x; 1.1649x over previous
import functools

import jax
import jax.numpy as jnp
from jax.experimental import pallas as pl
from jax.experimental.pallas import tpu as pltpu


_BLK = 8000
_NPART = 2
_WIN = 40
_OUT_S = 1024


def _pool_kernel(*refs, nparts, psteps, s_out, win):
    blkinfo_ref, b2_ref = refs[0], refs[1]
    batch_refs = refs[2:2 + nparts]
    x_refs = refs[2 + nparts:2 + 2 * nparts]
    w1_ref, b1_ref, w2_ref = refs[2 + 2 * nparts:5 + 2 * nparts]
    out_ref = refs[5 + 2 * nparts]
    scratch = refs[6 + 2 * nparts:]
    acc_refs = scratch[0:nparts]
    m_refs = scratch[nparts:2 * nparts]
    l_refs = scratch[2 * nparts:3 * nparts]

    k = pl.program_id(0)

    @pl.when(k == 0)
    def _init():
        for q in range(nparts):
            acc_refs[q][:, :] = jnp.zeros_like(acc_refs[q])
            m_refs[q][:, :] = jnp.zeros_like(m_refs[q])
            l_refs[q][:, :] = jnp.zeros_like(l_refs[q])

    w1_bf = w1_ref[:, :].astype(jnp.bfloat16)
    w2_bf = w2_ref[:, :].astype(jnp.bfloat16)
    b1_bf = b1_ref[:, :].astype(jnp.bfloat16)
    b2 = b2_ref[0]

    x_bfs, gates = [], []
    for q in range(nparts):
        x_bf = x_refs[q][:, :].astype(jnp.bfloat16)
        h = jax.lax.dot_general(x_bf, w1_bf, (((1,), (1,)), ((), ())),
                                preferred_element_type=jnp.float32)
        h = jnp.maximum(h.astype(jnp.bfloat16) + b1_bf, 0)
        gate = jax.lax.dot_general(w2_bf, h, (((1,), (1,)), ((), ())),
                                   preferred_element_type=jnp.float32)
        gates.append(gate + b2)
        x_bfs.append(x_bf)

    idss = [batch_refs[q][0] for q in range(nparts)]
    start0s = [blkinfo_ref[q * psteps + k, 0] for q in range(nparts)]
    nwins = [(blkinfo_ref[q * psteps + k, 1] - start0s[q]) // win + 1
             for q in range(nparts)]
    nwin_max = nwins[0]
    for q in range(1, nparts):
        nwin_max = jnp.maximum(nwin_max, nwins[q])

    def body(j, carry):
        for q in range(nparts):
            @pl.when(j < nwins[q])
            def _do(q=q):
                ids, gate, x_bf = idss[q], gates[q], x_bfs[q]
                acc_ref, m_ref, l_ref = acc_refs[q], m_refs[q], l_refs[q]
                start = start0s[q] + j * win
                rows = start + jax.lax.broadcasted_iota(
                    jnp.int32, (win, 1), 0)
                hit = ids == rows
                bmax = jnp.max(jnp.where(hit, gate, -1e30), axis=1,
                               keepdims=True)
                m_old = m_ref[pl.ds(start, win), :]
                m_new = jnp.maximum(m_old, bmax)
                alpha = jnp.exp(m_old - m_new)
                m_ref[pl.ds(start, win), :] = m_new
                m_row = jnp.sum(jnp.where(hit, m_new, 0.0), axis=0,
                                keepdims=True)
                in_w = (ids >= start) & (ids < start + win)
                e = jnp.where(in_w, jnp.exp(gate - m_row), 0.0)
                p = jnp.where(hit, e, 0.0)
                l_add = jnp.sum(p, axis=1, keepdims=True)
                l_ref[pl.ds(start, win), :] = (
                    l_ref[pl.ds(start, win), :] * alpha + l_add)
                contrib = jax.lax.dot_general(
                    p.astype(jnp.bfloat16), x_bf, (((1,), (0,)), ((), ())),
                    preferred_element_type=jnp.float32)
                acc_ref[pl.ds(start, win), :] = (
                    acc_ref[pl.ds(start, win), :] * alpha + contrib)
        return carry

    jax.lax.fori_loop(0, nwin_max, body, 0)

    @pl.when(k == psteps - 1)
    def _fin():
        m_parts = [m_refs[q][0:s_out, :] for q in range(nparts)]
        m_all = m_parts[0]
        for q in range(1, nparts):
            m_all = jnp.maximum(m_all, m_parts[q])
        num = jnp.zeros((s_out, out_ref.shape[1]), jnp.float32)
        den = jnp.zeros((s_out, 1), jnp.float32)
        for q in range(nparts):
            scale = jnp.exp(m_parts[q] - m_all)
            num = num + acc_refs[q][0:s_out, :] * scale
            den = den + l_refs[q][0:s_out, :] * scale
        out_ref[:, :] = num / (den + 1e-9)


def _run(x, batch, W1, b1, W2, b2, s_out, blk, nparts, win, interpret=False):
    n, d = x.shape
    hdim = W1.shape[0]
    nsteps = n // blk
    psteps = nsteps // nparts
    assert psteps * nparts * blk == n
    s_pad = s_out + 2 * win

    batch3 = batch.reshape(nsteps, 1, blk)
    first = (batch3[:, 0, 0] // 8) * 8
    last = batch3[:, 0, blk - 1]
    blkinfo = jnp.stack([first, last], axis=1).astype(jnp.int32)

    def batch_map(q):
        return lambda k: (q * psteps + k, 0, 0)

    def x_map(q):
        return lambda k: (q * psteps + k, 0)

    in_specs = (
        [pl.BlockSpec(memory_space=pltpu.SMEM),
         pl.BlockSpec(memory_space=pltpu.SMEM)]
        + [pl.BlockSpec((1, 1, blk), batch_map(q)) for q in range(nparts)]
        + [pl.BlockSpec((blk, d), x_map(q)) for q in range(nparts)]
        + [pl.BlockSpec((hdim, d), lambda k: (0, 0)),
           pl.BlockSpec((1, hdim), lambda k: (0, 0)),
           pl.BlockSpec((1, hdim), lambda k: (0, 0))]
    )

    scratch_shapes = (
        [pltpu.VMEM((s_pad, d), jnp.float32) for _ in range(nparts)]
        + [pltpu.VMEM((s_pad, 1), jnp.float32) for _ in range(2 * nparts)]
    )

    fn = pl.pallas_call(
        functools.partial(_pool_kernel, nparts=nparts, psteps=psteps,
                          s_out=s_out, win=win),
        grid=(psteps,),
        in_specs=in_specs,
        out_specs=pl.BlockSpec((s_out, d), lambda k: (0, 0)),
        out_shape=jax.ShapeDtypeStruct((s_out, d), x.dtype),
        scratch_shapes=scratch_shapes,
        compiler_params=pltpu.CompilerParams(
            dimension_semantics=("arbitrary",)),
        interpret=interpret,
    )
    args = ([blkinfo, b2] + [batch3] * nparts + [x] * nparts
            + [W1, b1.reshape(1, hdim), W2])
    return fn(*args)


def kernel(x, batch, W1, b1, W2, b2):
    return _run(x, batch, W1, b1, W2, b2, _OUT_S, _BLK, _NPART, _WIN)
